# baseline (speedup 1.0000x reference)
import functools
import jax
import jax.numpy as jnp
from jax import lax
from jax.experimental import pallas as pl
from jax.experimental.pallas import tpu as pltpu
from jax.experimental.pallas import tpu_sc as plsc

N = 10000
E = 320000
D = 128
H = 256
GP = 210
GPP = 256
G = 128

_NC = 2
_NS = 16
_B = 80
_CPP = 63
_EPP = _B * _CPP
E_PAD = _NC * _NS * 2 * _EPP
N_ACC = N + 8


def _sc_mesh():
    return plsc.VectorSubcoreMesh(core_axis_name="c", subcore_axis_name="s",
                                  num_cores=_NC, num_subcores=_NS)


def _k1_body(dst_hbm, ones_hbm, zeros_hbm, out_hbm, idx_v, ones_v, acc_sh):
    c = lax.axis_index("c")
    s = lax.axis_index("s")
    pltpu.sync_copy(ones_hbm, ones_v)

    @pl.when(s == 0)
    def _():
        pltpu.sync_copy(zeros_hbm, acc_sh)

    plsc.subcore_barrier()

    for p in range(2):
        pltpu.sync_copy(dst_hbm.at[c, s, p], idx_v)

        def chunk(k, carry):
            pltpu.sync_copy(ones_v, acc_sh.at[idx_v.at[k]], add=True)
            return carry

        lax.fori_loop(0, _CPP, chunk, 0)

    plsc.subcore_barrier()

    @pl.when(s == 0)
    def _():
        pltpu.sync_copy(acc_sh, out_hbm.at[c])


def _k1(dst_r):
    ones = jnp.ones((_B,), jnp.float32)
    zn = jnp.zeros((N_ACC,), jnp.float32)
    return pl.kernel(
        _k1_body,
        out_type=jax.ShapeDtypeStruct((_NC, N_ACC), jnp.float32),
        mesh=_sc_mesh(),
        scratch_types=[
            pltpu.VMEM((_CPP, _B), jnp.int32),
            pltpu.VMEM((_B,), jnp.float32),
            pltpu.VMEM_SHARED((N_ACC,), jnp.float32),
        ],
    )(dst_r, ones, zn)


def _seg_body(nphases, tab_hbm, src_hbm, dst_hbm, zeros_hbm, out_hbm,
              sidx_v, didx_v, b0, b1, b2, acc_sh,
              g0, g1, g2, s0, s1, s2):
    c = lax.axis_index("c")
    s = lax.axis_index("s")
    bufs = (b0, b1, b2)
    gsems = (g0, g1, g2)
    ssems = (s0, s1, s2)

    @pl.when(s == 0)
    def _():
        pltpu.sync_copy(zeros_hbm, acc_sh)

    plsc.subcore_barrier()

    for p in range(nphases):
        pltpu.sync_copy(src_hbm.at[c, s, p], sidx_v)
        pltpu.sync_copy(dst_hbm.at[c, s, p], didx_v)

        pltpu.async_copy(tab_hbm.at[sidx_v.at[0]], bufs[0], gsems[0])
        pltpu.async_copy(tab_hbm.at[sidx_v.at[1]], bufs[1], gsems[1])

        def triple(g, carry):
            for i in range(3):
                k = 3 * g + i
                buf, gsem, ssem = bufs[i], gsems[i], ssems[i]
                j = (i + 2) % 3
                pltpu.make_async_copy(tab_hbm.at[sidx_v.at[k]], buf, gsem).wait()
                pltpu.async_copy(buf, acc_sh.at[didx_v.at[k]], ssem, add=True)

                @pl.when(k + 2 < _CPP)
                def _():
                    @pl.when(k >= 1)
                    def _():
                        pltpu.make_async_copy(
                            bufs[j], acc_sh.at[didx_v.at[k - 1]], ssems[j]).wait()

                    pltpu.async_copy(tab_hbm.at[sidx_v.at[k + 2]],
                                     bufs[j], gsems[j])

            return carry

        lax.fori_loop(0, _CPP // 3, triple, 0)
        for kk, i in ((_CPP - 3, 0), (_CPP - 2, 1), (_CPP - 1, 2)):
            pltpu.make_async_copy(bufs[i], acc_sh.at[didx_v.at[kk]],
                                  ssems[i]).wait()

    plsc.subcore_barrier()

    @pl.when(s == 0)
    def _():
        pltpu.sync_copy(acc_sh.at[pl.ds(0, N)], out_hbm.at[c])


def _seg(tab, src_r, dst_r, nphases):
    zeros = jnp.zeros((N_ACC, D), jnp.float32)
    body = functools.partial(_seg_body, nphases)
    return pl.kernel(
        body,
        out_type=jax.ShapeDtypeStruct((_NC, N, D), jnp.float32),
        mesh=_sc_mesh(),
        scratch_types=[
            pltpu.VMEM((_CPP, _B), jnp.int32),
            pltpu.VMEM((_CPP, _B), jnp.int32),
            pltpu.VMEM((_B, D), jnp.float32),
            pltpu.VMEM((_B, D), jnp.float32),
            pltpu.VMEM((_B, D), jnp.float32),
            pltpu.VMEM_SHARED((N_ACC, D), jnp.float32),
            pltpu.SemaphoreType.DMA,
            pltpu.SemaphoreType.DMA,
            pltpu.SemaphoreType.DMA,
            pltpu.SemaphoreType.DMA,
            pltpu.SemaphoreType.DMA,
            pltpu.SemaphoreType.DMA,
        ],
    )(tab, src_r, dst_r, zeros)


_W7 = GPP // 2


def _k7_compute(k, a, b, o, out_hbm, rowbase):
    def one_edge(i):
        acc0 = None
        acc1 = None
        for j in range(_W7 // 16):
            va = plsc.bitcast(a[i, pl.ds(16 * j, 16)], jnp.bfloat16)
            vb = plsc.bitcast(b[i, pl.ds(16 * j, 16)], jnp.bfloat16)
            prod = va * vb
            lo, hi = plsc.unpack(prod, format=plsc.PackFormat.INTERLEAVED,
                                 preferred_element_type=jnp.float32)
            if acc0 is None:
                acc0, acc1 = lo, hi
            else:
                acc0 = acc0 + lo
                acc1 = acc1 + hi
        o[i] = acc0 + acc1

    def edge(e, carry):
        i = 2 * e
        one_edge(i)
        one_edge(i + 1)
        return carry

    lax.fori_loop(0, _B // 2, edge, 0)
    pltpu.sync_copy(o, out_hbm.at[pl.ds(rowbase + k * _B, _B)])


def _k7_body(tab_hbm, src_hbm, dst_hbm, out_hbm,
             sidx_v, didx_v, a0, a1, a2, b0, b1, b2, o_v,
             ga0, ga1, ga2, gb0, gb1, gb2):
    c = lax.axis_index("c")
    s = lax.axis_index("s")
    abufs = (a0, a1, a2)
    bbufs = (b0, b1, b2)
    gas = (ga0, ga1, ga2)
    gbs = (gb0, gb1, gb2)

    for p in range(2):
        rowbase = ((c * _NS + s) * 2 + p) * _EPP
        pltpu.sync_copy(src_hbm.at[c, s, p], sidx_v)
        pltpu.sync_copy(dst_hbm.at[c, s, p], didx_v)

        for i in range(2):
            pltpu.async_copy(tab_hbm.at[sidx_v.at[i]], abufs[i], gas[i])
            pltpu.async_copy(tab_hbm.at[didx_v.at[i]], bbufs[i], gbs[i])

        def triple(g, carry):
            for i in range(3):
                k = 3 * g + i
                j = (i + 2) % 3
                pltpu.make_async_copy(tab_hbm.at[sidx_v.at[k]],
                                      abufs[i], gas[i]).wait()
                pltpu.make_async_copy(tab_hbm.at[didx_v.at[k]],
                                      bbufs[i], gbs[i]).wait()
                _k7_compute(k, abufs[i], bbufs[i], o_v, out_hbm, rowbase)

                @pl.when(k + 2 < _CPP)
                def _():
                    pltpu.async_copy(tab_hbm.at[sidx_v.at[k + 2]],
                                     abufs[j], gas[j])
                    pltpu.async_copy(tab_hbm.at[didx_v.at[k + 2]],
                                     bbufs[j], gbs[j])

            return carry

        lax.fori_loop(0, _CPP // 3, triple, 0)


def _k7(mup32, src_r, dst_r):
    return pl.kernel(
        _k7_body,
        out_type=jax.ShapeDtypeStruct((E_PAD, 16), jnp.float32),
        mesh=_sc_mesh(),
        compiler_params=pltpu.CompilerParams(needs_layout_passes=False),
        scratch_types=[
            pltpu.VMEM((_CPP, _B), jnp.int32),
            pltpu.VMEM((_CPP, _B), jnp.int32),
            pltpu.VMEM((_B, _W7), jnp.int32),
            pltpu.VMEM((_B, _W7), jnp.int32),
            pltpu.VMEM((_B, _W7), jnp.int32),
            pltpu.VMEM((_B, _W7), jnp.int32),
            pltpu.VMEM((_B, _W7), jnp.int32),
            pltpu.VMEM((_B, _W7), jnp.int32),
            pltpu.VMEM((_B, 16), jnp.float32),
            pltpu.SemaphoreType.DMA,
            pltpu.SemaphoreType.DMA,
            pltpu.SemaphoreType.DMA,
            pltpu.SemaphoreType.DMA,
            pltpu.SemaphoreType.DMA,
            pltpu.SemaphoreType.DMA,
        ],
    )(mup32, src_r, dst_r)


def _k2_body(deg2_ref, x_ref, wdec_ref, mask_ref, src_ref,
             dinv_ref, xls_ref, wdm_ref, idx2_ref):
    deg = deg2_ref[0, :N] + deg2_ref[1, :N]
    deg = jnp.maximum(deg, 1.0)
    dinv = jax.lax.rsqrt(deg)
    dinv_ref[...] = dinv.reshape(N, 1)
    xls_ref[...] = jnp.log1p(x_ref[...]) * dinv.reshape(N, 1)
    wdm_ref[...] = wdec_ref[...] * mask_ref[...]
    s2 = src_ref[...] * 2
    idx2_ref[...] = jnp.stack([s2, s2 + 1])


def _k2(deg2, x, wdec, mask, src2d):
    return pl.pallas_call(
        _k2_body,
        out_shape=(
            jax.ShapeDtypeStruct((N, 1), jnp.float32),
            jax.ShapeDtypeStruct((N, D), jnp.float32),
            jax.ShapeDtypeStruct((GP, G), jnp.float32),
            jax.ShapeDtypeStruct((2, E_PAD // G, G), jnp.int32),
        ),
    )(deg2, x, wdec, mask, src2d)


_BN4 = 2000


def _k4_body(seg_ref, dinv_ref, w1_ref, b1_ref, hs_ref):
    s = seg_ref[0] + seg_ref[1]
    dv = dinv_ref[...]
    y = s * dv
    h = jnp.maximum(jnp.dot(y, w1_ref[...],
                            preferred_element_type=jnp.float32) + b1_ref[...], 0.0)
    hs_ref[...] = h * dv


def _k4(seg1, dinv, w1, b1):
    grid = (N // _BN4,)
    return pl.pallas_call(
        _k4_body,
        grid=grid,
        in_specs=[
            pl.BlockSpec((2, _BN4, D), lambda i: (0, i, 0)),
            pl.BlockSpec((_BN4, 1), lambda i: (i, 0)),
            pl.BlockSpec((D, H), lambda i: (0, 0)),
            pl.BlockSpec((1, H), lambda i: (0, 0)),
        ],
        out_specs=pl.BlockSpec((_BN4, H), lambda i: (i, 0)),
        out_shape=jax.ShapeDtypeStruct((N, H), jnp.float32),
    )(seg1, dinv, w1, b1)


_BN6 = 2000


def _k6a_body(seg_ref, dinv_ref, wmua_ref, wmub_ref, mu_ref, mup_ref):
    dv = dinv_ref[...]
    h2a = seg_ref[0] * dv
    h2b = seg_ref[1] * dv
    mu = (jnp.dot(h2a, wmua_ref[...], preferred_element_type=jnp.float32)
          + jnp.dot(h2b, wmub_ref[...], preferred_element_type=jnp.float32))
    mu_ref[...] = mu
    mub = jnp.pad(mu, ((0, 0), (0, GPP - GP))).astype(jnp.bfloat16)
    lo = lax.bitcast_convert_type(mub[:, :_W7], jnp.uint16).astype(jnp.uint32)
    hi = lax.bitcast_convert_type(mub[:, _W7:], jnp.uint16).astype(jnp.uint32)
    mup_ref[...] = ((hi << 16) | lo).astype(jnp.int32)


def _k6a(seg2, dinv, wmu):
    grid = (N // _BN6,)
    return pl.pallas_call(
        _k6a_body,
        grid=grid,
        in_specs=[
            pl.BlockSpec((2, _BN6, D), lambda i: (0, i, 0)),
            pl.BlockSpec((_BN6, 1), lambda i: (i, 0)),
            pl.BlockSpec((D, GP), lambda i: (0, 0)),
            pl.BlockSpec((H - D, GP), lambda i: (0, 0)),
        ],
        out_specs=(
            pl.BlockSpec((_BN6, GP), lambda i: (i, 0)),
            pl.BlockSpec((_BN6, _W7), lambda i: (i, 0)),
        ),
        out_shape=(
            jax.ShapeDtypeStruct((N, GP), jnp.float32),
            jax.ShapeDtypeStruct((N, _W7), jnp.int32),
        ),
    )(seg2, dinv, wmu[:D], wmu[D:])


def _k6b_body(seg_ref, dinv_ref, wloa_ref, wlob_ref, wdm_ref, mu_ref,
              lo_ref, nb_ref):
    dv = dinv_ref[...]
    h2a = seg_ref[0] * dv
    h2b = seg_ref[1] * dv
    lo = (jnp.dot(h2a, wloa_ref[...], preferred_element_type=jnp.float32)
          + jnp.dot(h2b, wlob_ref[...], preferred_element_type=jnp.float32))
    lo_ref[...] = lo
    nb_ref[...] = jnp.exp(jnp.clip(
        jnp.dot(mu_ref[...], wdm_ref[...], preferred_element_type=jnp.float32),
        -10.0, 10.0))


def _k6b(seg2, dinv, wlo, wdm, mu):
    grid = (N // _BN6,)
    return pl.pallas_call(
        _k6b_body,
        grid=grid,
        in_specs=[
            pl.BlockSpec((2, _BN6, D), lambda i: (0, i, 0)),
            pl.BlockSpec((_BN6, 1), lambda i: (i, 0)),
            pl.BlockSpec((D, GP), lambda i: (0, 0)),
            pl.BlockSpec((H - D, GP), lambda i: (0, 0)),
            pl.BlockSpec((GP, G), lambda i: (0, 0)),
            pl.BlockSpec((_BN6, GP), lambda i: (i, 0)),
        ],
        out_specs=(
            pl.BlockSpec((_BN6, GP), lambda i: (i, 0)),
            pl.BlockSpec((_BN6, G), lambda i: (i, 0)),
        ),
        out_shape=(
            jax.ShapeDtypeStruct((N, GP), jnp.float32),
            jax.ShapeDtypeStruct((N, G), jnp.float32),
        ),
    )(seg2, dinv, wlo[:D], wlo[D:], wdm, mu)


_BE8 = E_PAD // 16


def _k8_body(p_ref, o_ref):
    o_ref[...] = jnp.sum(p_ref[...], axis=1, keepdims=True)


def _k8(part):
    grid = (E_PAD // _BE8,)
    return pl.pallas_call(
        _k8_body,
        grid=grid,
        in_specs=[pl.BlockSpec((_BE8, 16), lambda i: (i, 0))],
        out_specs=pl.BlockSpec((_BE8, 1), lambda i: (i, 0)),
        out_shape=jax.ShapeDtypeStruct((E_PAD, 1), jnp.float32),
    )(part)


def kernel(x, edge_index, W1, b1, W_mu, W_logstd, W_dec, mask):
    src = edge_index[0].astype(jnp.int32)
    dst = edge_index[1].astype(jnp.int32)
    fill = (jnp.arange(E_PAD - E, dtype=jnp.int32) % 8)
    src_p = jnp.concatenate([src, fill])
    dst_p = jnp.concatenate([dst, N + fill])
    src_r = src_p.reshape(_NC, _NS, 2, _CPP, _B)
    dst_r = dst_p.reshape(_NC, _NS, 2, _CPP, _B)

    deg2 = _k1(dst_r)
    dinv, xls, wdm, idx2 = _k2(deg2, x, W_dec, mask,
                               src_p.reshape(E_PAD // G, G))

    seg1 = _seg(xls, src_r, dst_r, 2)
    hs = _k4(seg1, dinv, W1, b1.reshape(1, H))

    idx2_r = idx2.reshape(_NC, _NS, 4, _CPP, _B)
    dst_r5 = jnp.broadcast_to(dst_p.reshape(1, _NS, 4, _CPP, _B),
                              (_NC, _NS, 4, _CPP, _B))
    seg2 = _seg(hs.reshape(2 * N, D), idx2_r, dst_r5, 4)

    mu, mup32 = _k6a(seg2, dinv, W_mu)
    part = _k7(mup32, src_r, dst_r)
    logstd, nb_mean = _k6b(seg2, dinv, W_logstd, wdm, mu)
    logits = _k8(part).reshape(E_PAD)[:E]

    return (nb_mean, logits, mu, logstd)


# speedup vs baseline: 1.1202x; 1.1202x over previous
import functools
import jax
import jax.numpy as jnp
from jax import lax
from jax.experimental import pallas as pl
from jax.experimental.pallas import tpu as pltpu
from jax.experimental.pallas import tpu_sc as plsc

N = 10000
E = 320000
D = 128
H = 256
GP = 210
GPP = 256
G = 128

_NC = 2
_NS = 16
_B = 80
_CPP = 63
_EPP = _B * _CPP
E_PAD = _NC * _NS * 2 * _EPP
N_ACC = N + 8


def _sc_mesh():
    return plsc.VectorSubcoreMesh(core_axis_name="c", subcore_axis_name="s",
                                  num_cores=_NC, num_subcores=_NS)


def _k1_body(dst_hbm, ones_hbm, zeros_hbm, out_hbm, idx_v, ones_v, acc_sh):
    c = lax.axis_index("c")
    s = lax.axis_index("s")
    pltpu.sync_copy(ones_hbm, ones_v)

    @pl.when(s == 0)
    def _():
        pltpu.sync_copy(zeros_hbm, acc_sh)

    plsc.subcore_barrier()

    for p in range(2):
        pltpu.sync_copy(dst_hbm.at[c, s, p], idx_v)

        def chunk(k, carry):
            pltpu.sync_copy(ones_v, acc_sh.at[idx_v.at[k]], add=True)
            return carry

        lax.fori_loop(0, _CPP, chunk, 0)

    plsc.subcore_barrier()

    @pl.when(s == 0)
    def _():
        pltpu.sync_copy(acc_sh, out_hbm.at[c])


def _k1(dst_r):
    ones = jnp.ones((_B,), jnp.float32)
    zn = jnp.zeros((N_ACC,), jnp.float32)
    return pl.kernel(
        _k1_body,
        out_type=jax.ShapeDtypeStruct((_NC, N_ACC), jnp.float32),
        mesh=_sc_mesh(),
        scratch_types=[
            pltpu.VMEM((_CPP, _B), jnp.int32),
            pltpu.VMEM((_B,), jnp.float32),
            pltpu.VMEM_SHARED((N_ACC,), jnp.float32),
        ],
    )(dst_r, ones, zn)


def _seg_body(nphases, tab_hbm, src_hbm, dst_hbm, zeros_hbm, out_hbm,
              sidx_v, didx_v, b0, b1, b2, acc_sh,
              g0, g1, g2, s0, s1, s2):
    c = lax.axis_index("c")
    s = lax.axis_index("s")
    bufs = (b0, b1, b2)
    gsems = (g0, g1, g2)
    ssems = (s0, s1, s2)

    @pl.when(s == 0)
    def _():
        pltpu.sync_copy(zeros_hbm, acc_sh)

    plsc.subcore_barrier()

    for p in range(nphases):
        pltpu.sync_copy(src_hbm.at[c, s, p], sidx_v)
        pltpu.sync_copy(dst_hbm.at[c, s, p], didx_v)

        pltpu.async_copy(tab_hbm.at[sidx_v.at[0]], bufs[0], gsems[0])
        pltpu.async_copy(tab_hbm.at[sidx_v.at[1]], bufs[1], gsems[1])

        def triple(g, carry):
            for i in range(3):
                k = 3 * g + i
                buf, gsem, ssem = bufs[i], gsems[i], ssems[i]
                j = (i + 2) % 3
                pltpu.make_async_copy(tab_hbm.at[sidx_v.at[k]], buf, gsem).wait()
                pltpu.async_copy(buf, acc_sh.at[didx_v.at[k]], ssem, add=True)

                @pl.when(k + 2 < _CPP)
                def _():
                    @pl.when(k >= 1)
                    def _():
                        pltpu.make_async_copy(
                            bufs[j], acc_sh.at[didx_v.at[k - 1]], ssems[j]).wait()

                    pltpu.async_copy(tab_hbm.at[sidx_v.at[k + 2]],
                                     bufs[j], gsems[j])

            return carry

        lax.fori_loop(0, _CPP // 3, triple, 0)
        for kk, i in ((_CPP - 3, 0), (_CPP - 2, 1), (_CPP - 1, 2)):
            pltpu.make_async_copy(bufs[i], acc_sh.at[didx_v.at[kk]],
                                  ssems[i]).wait()

    plsc.subcore_barrier()

    @pl.when(s == 0)
    def _():
        pltpu.sync_copy(acc_sh.at[pl.ds(0, N)], out_hbm.at[c])


def _seg(tab, src_r, dst_r, nphases):
    zeros = jnp.zeros((N_ACC, D), jnp.float32)
    body = functools.partial(_seg_body, nphases)
    return pl.kernel(
        body,
        out_type=jax.ShapeDtypeStruct((_NC, N, D), jnp.float32),
        mesh=_sc_mesh(),
        scratch_types=[
            pltpu.VMEM((_CPP, _B), jnp.int32),
            pltpu.VMEM((_CPP, _B), jnp.int32),
            pltpu.VMEM((_B, D), jnp.float32),
            pltpu.VMEM((_B, D), jnp.float32),
            pltpu.VMEM((_B, D), jnp.float32),
            pltpu.VMEM_SHARED((N_ACC, D), jnp.float32),
            pltpu.SemaphoreType.DMA,
            pltpu.SemaphoreType.DMA,
            pltpu.SemaphoreType.DMA,
            pltpu.SemaphoreType.DMA,
            pltpu.SemaphoreType.DMA,
            pltpu.SemaphoreType.DMA,
        ],
    )(tab, src_r, dst_r, zeros)


_W7 = GPP // 2


def _k7_compute(k, a, b, o, out_hbm, rowbase):
    lanes = lax.iota(jnp.int32, 16)

    def one_edge(i):
        acc0 = None
        acc1 = None
        for j in range(_W7 // 16):
            va = plsc.bitcast(a[i, pl.ds(16 * j, 16)], jnp.bfloat16)
            vb = plsc.bitcast(b[i, pl.ds(16 * j, 16)], jnp.bfloat16)
            prod = va * vb
            lo, hi = plsc.unpack(prod, format=plsc.PackFormat.INTERLEAVED,
                                 preferred_element_type=jnp.float32)
            if acc0 is None:
                acc0, acc1 = lo, hi
            else:
                acc0 = acc0 + lo
                acc1 = acc1 + hi
        return jnp.sum(acc0 + acc1)

    def group(gi, carry):
        ovec = jnp.zeros((16,), jnp.float32)
        for l in range(16):
            v = one_edge(16 * gi + l)
            ovec = jnp.where(lanes == l, v, ovec)
        o[pl.ds(16 * gi, 16)] = ovec
        return carry

    lax.fori_loop(0, _B // 16, group, 0)
    pltpu.sync_copy(o, out_hbm.at[pl.ds(rowbase + k * _B, _B)])


def _k7_body(tab_hbm, src_hbm, dst_hbm, out_hbm,
             sidx_v, didx_v, a0, a1, a2, b0, b1, b2, o_v,
             ga0, ga1, ga2, gb0, gb1, gb2):
    c = lax.axis_index("c")
    s = lax.axis_index("s")
    abufs = (a0, a1, a2)
    bbufs = (b0, b1, b2)
    gas = (ga0, ga1, ga2)
    gbs = (gb0, gb1, gb2)

    for p in range(2):
        rowbase = ((c * _NS + s) * 2 + p) * _EPP
        pltpu.sync_copy(src_hbm.at[c, s, p], sidx_v)
        pltpu.sync_copy(dst_hbm.at[c, s, p], didx_v)

        for i in range(2):
            pltpu.async_copy(tab_hbm.at[sidx_v.at[i]], abufs[i], gas[i])
            pltpu.async_copy(tab_hbm.at[didx_v.at[i]], bbufs[i], gbs[i])

        def triple(g, carry):
            for i in range(3):
                k = 3 * g + i
                j = (i + 2) % 3
                pltpu.make_async_copy(tab_hbm.at[sidx_v.at[k]],
                                      abufs[i], gas[i]).wait()
                pltpu.make_async_copy(tab_hbm.at[didx_v.at[k]],
                                      bbufs[i], gbs[i]).wait()
                _k7_compute(k, abufs[i], bbufs[i], o_v, out_hbm, rowbase)

                @pl.when(k + 2 < _CPP)
                def _():
                    pltpu.async_copy(tab_hbm.at[sidx_v.at[k + 2]],
                                     abufs[j], gas[j])
                    pltpu.async_copy(tab_hbm.at[didx_v.at[k + 2]],
                                     bbufs[j], gbs[j])

            return carry

        lax.fori_loop(0, _CPP // 3, triple, 0)


def _k7(mup32, src_r, dst_r):
    return pl.kernel(
        _k7_body,
        out_type=jax.ShapeDtypeStruct((E_PAD,), jnp.float32),
        mesh=_sc_mesh(),
        compiler_params=pltpu.CompilerParams(needs_layout_passes=False),
        scratch_types=[
            pltpu.VMEM((_CPP, _B), jnp.int32),
            pltpu.VMEM((_CPP, _B), jnp.int32),
            pltpu.VMEM((_B, _W7), jnp.int32),
            pltpu.VMEM((_B, _W7), jnp.int32),
            pltpu.VMEM((_B, _W7), jnp.int32),
            pltpu.VMEM((_B, _W7), jnp.int32),
            pltpu.VMEM((_B, _W7), jnp.int32),
            pltpu.VMEM((_B, _W7), jnp.int32),
            pltpu.VMEM((_B,), jnp.float32),
            pltpu.SemaphoreType.DMA,
            pltpu.SemaphoreType.DMA,
            pltpu.SemaphoreType.DMA,
            pltpu.SemaphoreType.DMA,
            pltpu.SemaphoreType.DMA,
            pltpu.SemaphoreType.DMA,
        ],
    )(mup32, src_r, dst_r)


def _k2_body(deg2_ref, x_ref, wdec_ref, mask_ref, src_ref,
             dinv_ref, xls_ref, wdm_ref, idx2_ref):
    deg = deg2_ref[0, :N] + deg2_ref[1, :N]
    deg = jnp.maximum(deg, 1.0)
    dinv = jax.lax.rsqrt(deg)
    dinv_ref[...] = dinv.reshape(N, 1)
    xls_ref[...] = jnp.log1p(x_ref[...]) * dinv.reshape(N, 1)
    wdm_ref[...] = wdec_ref[...] * mask_ref[...]
    s2 = src_ref[...] * 2
    idx2_ref[...] = jnp.stack([s2, s2 + 1])


def _k2(deg2, x, wdec, mask, src2d):
    return pl.pallas_call(
        _k2_body,
        out_shape=(
            jax.ShapeDtypeStruct((N, 1), jnp.float32),
            jax.ShapeDtypeStruct((N, D), jnp.float32),
            jax.ShapeDtypeStruct((GP, G), jnp.float32),
            jax.ShapeDtypeStruct((2, E_PAD // G, G), jnp.int32),
        ),
    )(deg2, x, wdec, mask, src2d)


_BN4 = 2000


def _k4_body(seg_ref, dinv_ref, w1_ref, b1_ref, hs_ref):
    s = seg_ref[0] + seg_ref[1]
    dv = dinv_ref[...]
    y = s * dv
    h = jnp.maximum(jnp.dot(y, w1_ref[...],
                            preferred_element_type=jnp.float32) + b1_ref[...], 0.0)
    hs_ref[...] = h * dv


def _k4(seg1, dinv, w1, b1):
    grid = (N // _BN4,)
    return pl.pallas_call(
        _k4_body,
        grid=grid,
        in_specs=[
            pl.BlockSpec((2, _BN4, D), lambda i: (0, i, 0)),
            pl.BlockSpec((_BN4, 1), lambda i: (i, 0)),
            pl.BlockSpec((D, H), lambda i: (0, 0)),
            pl.BlockSpec((1, H), lambda i: (0, 0)),
        ],
        out_specs=pl.BlockSpec((_BN4, H), lambda i: (i, 0)),
        out_shape=jax.ShapeDtypeStruct((N, H), jnp.float32),
    )(seg1, dinv, w1, b1)


_BN6 = 2000


def _k6_body(seg_ref, dinv_ref, wmua_ref, wmub_ref, wloa_ref, wlob_ref,
             wdm_ref, mu_ref, mup_ref, lo_ref, nb_ref):
    dv = dinv_ref[...]
    h2a = seg_ref[0] * dv
    h2b = seg_ref[1] * dv
    mu = (jnp.dot(h2a, wmua_ref[...], preferred_element_type=jnp.float32)
          + jnp.dot(h2b, wmub_ref[...], preferred_element_type=jnp.float32))
    mu_ref[...] = mu
    mub = jnp.pad(mu, ((0, 0), (0, GPP - GP))).astype(jnp.bfloat16)
    plo = lax.bitcast_convert_type(mub[:, :_W7], jnp.uint16).astype(jnp.uint32)
    phi = lax.bitcast_convert_type(mub[:, _W7:], jnp.uint16).astype(jnp.uint32)
    mup_ref[...] = ((phi << 16) | plo).astype(jnp.int32)
    lo = (jnp.dot(h2a, wloa_ref[...], preferred_element_type=jnp.float32)
          + jnp.dot(h2b, wlob_ref[...], preferred_element_type=jnp.float32))
    lo_ref[...] = lo
    nb_ref[...] = jnp.exp(jnp.clip(
        jnp.dot(mu, wdm_ref[...], preferred_element_type=jnp.float32),
        -10.0, 10.0))


def _k6(seg2, dinv, wmu, wlo, wdm):
    grid = (N // _BN6,)
    return pl.pallas_call(
        _k6_body,
        grid=grid,
        in_specs=[
            pl.BlockSpec((2, _BN6, D), lambda i: (0, i, 0)),
            pl.BlockSpec((_BN6, 1), lambda i: (i, 0)),
            pl.BlockSpec((D, GP), lambda i: (0, 0)),
            pl.BlockSpec((H - D, GP), lambda i: (0, 0)),
            pl.BlockSpec((D, GP), lambda i: (0, 0)),
            pl.BlockSpec((H - D, GP), lambda i: (0, 0)),
            pl.BlockSpec((GP, G), lambda i: (0, 0)),
        ],
        out_specs=(
            pl.BlockSpec((_BN6, GP), lambda i: (i, 0)),
            pl.BlockSpec((_BN6, _W7), lambda i: (i, 0)),
            pl.BlockSpec((_BN6, GP), lambda i: (i, 0)),
            pl.BlockSpec((_BN6, G), lambda i: (i, 0)),
        ),
        out_shape=(
            jax.ShapeDtypeStruct((N, GP), jnp.float32),
            jax.ShapeDtypeStruct((N, _W7), jnp.int32),
            jax.ShapeDtypeStruct((N, GP), jnp.float32),
            jax.ShapeDtypeStruct((N, G), jnp.float32),
        ),
    )(seg2, dinv, wmu[:D], wmu[D:], wlo[:D], wlo[D:], wdm)


def kernel(x, edge_index, W1, b1, W_mu, W_logstd, W_dec, mask):
    src = edge_index[0].astype(jnp.int32)
    dst = edge_index[1].astype(jnp.int32)
    fill = (jnp.arange(E_PAD - E, dtype=jnp.int32) % 8)
    src_p = jnp.concatenate([src, fill])
    dst_p = jnp.concatenate([dst, N + fill])
    src_r = src_p.reshape(_NC, _NS, 2, _CPP, _B)
    dst_r = dst_p.reshape(_NC, _NS, 2, _CPP, _B)

    deg2 = _k1(dst_r)
    dinv, xls, wdm, idx2 = _k2(deg2, x, W_dec, mask,
                               src_p.reshape(E_PAD // G, G))

    seg1 = _seg(xls, src_r, dst_r, 2)
    hs = _k4(seg1, dinv, W1, b1.reshape(1, H))

    idx2_r = idx2.reshape(_NC, _NS, 4, _CPP, _B)
    dst_r5 = jnp.broadcast_to(dst_p.reshape(1, _NS, 4, _CPP, _B),
                              (_NC, _NS, 4, _CPP, _B))
    seg2 = _seg(hs.reshape(2 * N, D), idx2_r, dst_r5, 4)

    mu, mup32, logstd, nb_mean = _k6(seg2, dinv, W_mu, W_logstd, wdm)
    logits = _k7(mup32, src_r, dst_r)[:E]

    return (nb_mean, logits, mu, logstd)
